# baseline (speedup 1.0000x reference)
import functools

import jax
import jax.numpy as jnp
from jax import lax
from jax.experimental import pallas as pl
from jax.experimental.pallas import tpu as pltpu
from jax.experimental.pallas import tpu_sc as plsc

B = 4096
D = 64
DFF = 2048
M = 50000
DD = D * D

NC = 2
NS = 16
NW = NC * NS
L = 16
BPW = B // NW
GROUPS = BPW // L

_sc_mesh = plsc.VectorSubcoreMesh(core_axis_name="c", subcore_axis_name="s")


HD = DD // 2
NT = 2 * GROUPS


@functools.partial(
    pl.kernel,
    out_type=jax.ShapeDtypeStruct((B, DD), jnp.float32),
    mesh=_sc_mesh,
    scratch_types=[
        pltpu.VMEM((BPW,), jnp.int32),
        pltpu.VMEM((2, L, HD), jnp.float32),
        pltpu.SemaphoreType.DMA,
        pltpu.SemaphoreType.DMA,
    ],
)
def _sc_gather(states_hbm, ids_hbm, out_hbm, idx_v, bufs, sem_i, sem_o):
  wid = lax.axis_index("s") * NC + lax.axis_index("c")
  base = wid * BPW
  pltpu.sync_copy(ids_hbm.at[pl.ds(base, BPW)], idx_v)
  ins, wrs = {}, {}

  def gather(t):
    g, h = divmod(t, 2)
    idxs = idx_v[pl.ds(g * L, L)] * 2 + h
    return pltpu.async_copy(states_hbm.at[idxs], bufs.at[t % 2], sem_i)

  ins[0] = gather(0)
  for t in range(NT):
    if t + 1 < NT:
      if t - 1 >= 0:
        wrs[t - 1].wait()
      ins[t + 1] = gather(t + 1)
    ins[t].wait()
    g, h = divmod(t, 2)
    wrs[t] = pltpu.async_copy(
        bufs.at[t % 2],
        out_hbm.at[pl.ds(base + g * L, L), pl.ds(h * HD, HD)], sem_o)
  wrs[NT - 2].wait()
  wrs[NT - 1].wait()


@functools.partial(
    pl.kernel,
    out_type=(),
    mesh=_sc_mesh,
    scratch_types=[
        pltpu.VMEM((BPW,), jnp.int32),
        pltpu.VMEM((BPW,), jnp.int32),
        pltpu.VMEM((2, L, HD), jnp.float32),
        pltpu.SemaphoreType.DMA,
        pltpu.SemaphoreType.DMA,
    ],
)
def _sc_scatter(new_hbm, ids_hbm, win_hbm, states_ref, idx_v, win_v, bufs,
                sem_i, sem_o):
  wid = lax.axis_index("s") * NC + lax.axis_index("c")
  base = wid * BPW
  pltpu.sync_copy(ids_hbm.at[pl.ds(base, BPW)], idx_v)
  pltpu.sync_copy(win_hbm.at[pl.ds(base, BPW)], win_v)
  ins, scs = {}, {}

  def gnew(t):
    g, h = divmod(t, 2)
    wins = win_v[pl.ds(g * L, L)] * 2 + h
    return pltpu.async_copy(new_hbm.at[wins], bufs.at[t % 2], sem_i)

  ins[0] = gnew(0)
  for t in range(NT):
    if t + 1 < NT:
      if t - 1 >= 0:
        scs[t - 1].wait()
      ins[t + 1] = gnew(t + 1)
    ins[t].wait()
    g, h = divmod(t, 2)
    dsts = idx_v[pl.ds(g * L, L)] * 2 + h
    scs[t] = pltpu.async_copy(bufs.at[t % 2], states_ref.at[dsts], sem_o)
  scs[NT - 2].wait()
  scs[NT - 1].wait()


_sc_gather_ref = _sc_gather


WSZ = 256
WGRID = B // WSZ


def _win_body(ids_row_ref, ids_col_ref, win_ref):
  eq = ids_col_ref[...] == ids_row_ref[...]
  pos = lax.broadcasted_iota(jnp.int32, (WSZ, B), 1)
  win_ref[0, 0, :] = jnp.max(jnp.where(eq, pos, -1), axis=1)


def _win_idx(ids_row, ids_col):
  return pl.pallas_call(
      _win_body,
      grid=(WGRID,),
      in_specs=[
          pl.BlockSpec((1, B), lambda i: (0, 0)),
          pl.BlockSpec((WSZ, 1), lambda i: (i, 0)),
      ],
      out_specs=pl.BlockSpec((1, 1, WSZ), lambda i: (i, 0, 0)),
      out_shape=jax.ShapeDtypeStruct((WGRID, 1, WSZ), jnp.int32),
      compiler_params=pltpu.CompilerParams(
          dimension_semantics=("arbitrary",)),
  )(ids_row, ids_col)


BSZ = 128
GRID = B // BSZ


def _ln(x, g, b):
  mu = jnp.mean(x, axis=-1, keepdims=True)
  var = jnp.mean((x - mu) * (x - mu), axis=-1, keepdims=True)
  return (x - mu) * lax.rsqrt(var + 1e-5) * g + b


def _l2norm(x):
  n = jnp.sqrt(jnp.sum(x * x, axis=-1, keepdims=True))
  return x / jnp.maximum(n, 1e-12)


def _dot_t(a, b):
  return lax.dot_general(a, b, (((1,), (1,)), ((), ())),
                         preferred_element_type=jnp.float32)


def _dot(a, b):
  return lax.dot_general(a, b, (((1,), (0,)), ((), ())),
                         preferred_element_type=jnp.float32)


def _tc_body(use_ref, item_ref, old_ref, rep_ref, exp_ref, seg_ref,
             Wq_ref, Wk_ref, Wv_ref, Wo_ref, bo_ref, wa_ref, ba_ref,
             we_ref, be_ref, W1_ref, b1_ref, W2_ref, b2_ref,
             g1_ref, c1_ref, g2_ref, c2_ref,
             ude_ref, new_ref):
  use = use_ref[...]
  item = item_ref[...]
  old = old_ref[...]
  rep = rep_ref[...]
  expm = exp_ref[...]
  seg = seg_ref[...]
  g1, c1 = g1_ref[...], c1_ref[...]
  g2, c2 = g2_ref[...], c2_ref[...]

  x_norm = _ln(use, g1, c1)
  q = _l2norm(_dot_t(x_norm, Wq_ref[...]))
  qrep = _dot(q, rep)
  read_content = _dot(old * qrep, seg)
  attn = _dot_t(read_content, Wo_ref[...]) + bo_ref[...]
  x = use + attn
  x2 = _ln(x, g2, c2)
  h = jnp.maximum(_dot_t(x2, W1_ref[...]) + b1_ref[...], 0.0)
  ffn = _dot_t(h, W2_ref[...]) + b2_ref[...]
  ude_ref[...] = x + ffn

  i_norm = _ln(item, g1, c1)
  k = _l2norm(_dot_t(i_norm, Wk_ref[...]))
  v = _dot_t(i_norm, Wv_ref[...])
  alpha = jax.nn.sigmoid(
      jnp.sum(i_norm * wa_ref[...], axis=1, keepdims=True) + ba_ref[0, 0])
  eta = jax.nn.sigmoid(
      jnp.sum(i_norm * we_ref[...], axis=1, keepdims=True) + be_ref[0, 0]
  ) * (D ** -0.5)
  krep = _dot(k, rep)
  pred = _dot(old * krep, seg)
  err = v - pred
  errrep = _dot(err, expm)
  new_ref[...] = (1.0 - alpha) * old + eta * (errrep * krep)


def _tc_dense(use, item, old2, rep, expm, seg, Wq, Wk, Wv, Wout, bout,
              w_alpha, b_alpha, w_eta, b_eta, W1, b1, W2, b2,
              ln1_g, ln1_b, ln2_g, ln2_b):
  full = lambda s: pl.BlockSpec(s, lambda i: tuple(0 for _ in s))
  return pl.pallas_call(
      _tc_body,
      grid=(GRID,),
      in_specs=[
          pl.BlockSpec((BSZ, D), lambda i: (i, 0)),
          pl.BlockSpec((BSZ, D), lambda i: (i, 0)),
          pl.BlockSpec((BSZ, DD), lambda i: (i, 0)),
          full((D, DD)), full((D, DD)), full((DD, D)),
          full((D, D)), full((D, D)), full((D, D)), full((D, D)),
          full((1, D)),
          full((1, D)), full((1, 1)),
          full((1, D)), full((1, 1)),
          full((DFF, D)), full((1, DFF)),
          full((D, DFF)), full((1, D)),
          full((1, D)), full((1, D)), full((1, D)), full((1, D)),
      ],
      out_specs=[
          pl.BlockSpec((BSZ, D), lambda i: (i, 0)),
          pl.BlockSpec((BSZ, DD), lambda i: (i, 0)),
      ],
      out_shape=[
          jax.ShapeDtypeStruct((B, D), jnp.float32),
          jax.ShapeDtypeStruct((B, DD), jnp.float32),
      ],
      compiler_params=pltpu.CompilerParams(
          dimension_semantics=("arbitrary",)),
  )(use, item, old2, rep, expm, seg, Wq, Wk, Wv, Wout, bout,
    w_alpha, b_alpha, w_eta, b_eta, W1, b1, W2, b2,
    ln1_g, ln1_b, ln2_g, ln2_b)


def kernel(user_ids, user_static_emb, item_emb, states, Wq, Wk, Wv, Wout, bout,
           w_alpha, b_alpha, w_eta, b_eta, W1, b1, W2, b2,
           ln1_g, ln1_b, ln2_g, ln2_b):
  ids = user_ids.astype(jnp.int32)
  st_ref = jax.new_ref(states.reshape(2 * M, HD))

  old_flat = _sc_gather_ref(st_ref, ids)

  win3 = _win_idx(ids.reshape(1, B), ids.reshape(B, 1))

  pcol = lax.broadcasted_iota(jnp.int32, (D, DD), 1)
  prow = lax.broadcasted_iota(jnp.int32, (D, DD), 0)
  rep = (pcol % D == prow).astype(jnp.float32)
  expm = (pcol // D == prow).astype(jnp.float32)
  seg = expm.T

  ude, new2 = _tc_dense(
      user_static_emb, item_emb, old_flat, rep, expm, seg,
      Wq, Wk, Wv, Wout, bout.reshape(1, D),
      w_alpha, b_alpha.reshape(1, 1), w_eta, b_eta.reshape(1, 1),
      W1, b1.reshape(1, DFF), W2, b2.reshape(1, D),
      ln1_g.reshape(1, D), ln1_b.reshape(1, D),
      ln2_g.reshape(1, D), ln2_b.reshape(1, D))

  _sc_scatter(new2.reshape(2 * B, HD), ids, win3.reshape(B), st_ref)
  return ude, st_ref[...].reshape(M, D, D)


# speedup vs baseline: 2.6150x; 2.6150x over previous
import functools

import jax
import jax.numpy as jnp
from jax import lax
from jax.experimental import pallas as pl
from jax.experimental.pallas import tpu as pltpu
from jax.experimental.pallas import tpu_sc as plsc

B = 4096
D = 64
DFF = 2048
M = 50000
DD = D * D

NC = 2
NS = 16
NW = NC * NS
L = 16
BPW = B // NW
GROUPS = BPW // L

_sc_mesh = plsc.VectorSubcoreMesh(core_axis_name="c", subcore_axis_name="s")


@functools.partial(
    pl.kernel,
    out_type=jax.ShapeDtypeStruct((B, DD), jnp.float32),
    mesh=_sc_mesh,
    scratch_types=[
        pltpu.VMEM((BPW,), jnp.int32),
        pltpu.VMEM((L, DD), jnp.float32),
        pltpu.SemaphoreType.DMA,
    ],
)
def _sc_gather(states_hbm, ids_hbm, out_hbm, idx_v, rows_v, sem):
  wid = lax.axis_index("s") * NC + lax.axis_index("c")
  base = wid * BPW
  pltpu.sync_copy(ids_hbm.at[pl.ds(base, BPW)], idx_v)
  for g in range(GROUPS):
    idxs = idx_v[pl.ds(g * L, L)]
    pltpu.async_copy(states_hbm.at[idxs], rows_v, sem).wait()
    pltpu.sync_copy(rows_v, out_hbm.at[pl.ds(base + g * L, L)])


@functools.partial(
    pl.kernel,
    out_type=(),
    mesh=_sc_mesh,
    scratch_types=[
        pltpu.VMEM((BPW,), jnp.int32),
        pltpu.VMEM((BPW,), jnp.int32),
        pltpu.VMEM((L, DD), jnp.float32),
        pltpu.SemaphoreType.DMA,
    ],
)
def _sc_scatter(new_hbm, ids_hbm, win_hbm, states_ref, idx_v, win_v, rows_v, sem):
  wid = lax.axis_index("s") * NC + lax.axis_index("c")
  base = wid * BPW
  pltpu.sync_copy(ids_hbm.at[pl.ds(base, BPW)], idx_v)
  pltpu.sync_copy(win_hbm.at[pl.ds(base, BPW)], win_v)
  for g in range(GROUPS):
    wins = win_v[pl.ds(g * L, L)]
    pltpu.async_copy(new_hbm.at[wins], rows_v, sem).wait()
    dsts = idx_v[pl.ds(g * L, L)]
    pltpu.async_copy(rows_v, states_ref.at[dsts], sem).wait()


_sc_gather_ref = _sc_gather


WSZ = 256
WGRID = B // WSZ


def _win_body(ids_row_ref, ids_col_ref, win_ref):
  eq = ids_col_ref[...] == ids_row_ref[...]
  pos = lax.broadcasted_iota(jnp.int32, (WSZ, B), 1)
  win_ref[0, 0, :] = jnp.max(jnp.where(eq, pos, -1), axis=1)


def _win_idx(ids_row, ids_col):
  return pl.pallas_call(
      _win_body,
      grid=(WGRID,),
      in_specs=[
          pl.BlockSpec((1, B), lambda i: (0, 0)),
          pl.BlockSpec((WSZ, 1), lambda i: (i, 0)),
      ],
      out_specs=pl.BlockSpec((1, 1, WSZ), lambda i: (i, 0, 0)),
      out_shape=jax.ShapeDtypeStruct((WGRID, 1, WSZ), jnp.int32),
      compiler_params=pltpu.CompilerParams(
          dimension_semantics=("arbitrary",)),
  )(ids_row, ids_col)


BSZ = 128
GRID = B // BSZ


def _ln(x, g, b):
  mu = jnp.mean(x, axis=-1, keepdims=True)
  var = jnp.mean((x - mu) * (x - mu), axis=-1, keepdims=True)
  return (x - mu) * lax.rsqrt(var + 1e-5) * g + b


def _l2norm(x):
  n = jnp.sqrt(jnp.sum(x * x, axis=-1, keepdims=True))
  return x / jnp.maximum(n, 1e-12)


def _dot_t(a, b):
  return lax.dot_general(a, b, (((1,), (1,)), ((), ())),
                         preferred_element_type=jnp.float32)


def _dot(a, b):
  return lax.dot_general(a, b, (((1,), (0,)), ((), ())),
                         preferred_element_type=jnp.float32)


def _tc_body(use_ref, item_ref, old_ref, rep_ref, exp_ref, seg_ref,
             Wq_ref, Wk_ref, Wv_ref, Wo_ref, bo_ref, wa_ref, ba_ref,
             we_ref, be_ref, W1_ref, b1_ref, W2_ref, b2_ref,
             g1_ref, c1_ref, g2_ref, c2_ref,
             ude_ref, new_ref):
  use = use_ref[...]
  item = item_ref[...]
  old = old_ref[...]
  rep = rep_ref[...]
  expm = exp_ref[...]
  seg = seg_ref[...]
  g1, c1 = g1_ref[...], c1_ref[...]
  g2, c2 = g2_ref[...], c2_ref[...]

  x_norm = _ln(use, g1, c1)
  q = _l2norm(_dot_t(x_norm, Wq_ref[...]))
  qrep = _dot(q, rep)
  read_content = _dot(old * qrep, seg)
  attn = _dot_t(read_content, Wo_ref[...]) + bo_ref[...]
  x = use + attn
  x2 = _ln(x, g2, c2)
  h = jnp.maximum(_dot_t(x2, W1_ref[...]) + b1_ref[...], 0.0)
  ffn = _dot_t(h, W2_ref[...]) + b2_ref[...]
  ude_ref[...] = x + ffn

  i_norm = _ln(item, g1, c1)
  k = _l2norm(_dot_t(i_norm, Wk_ref[...]))
  v = _dot_t(i_norm, Wv_ref[...])
  alpha = jax.nn.sigmoid(
      jnp.sum(i_norm * wa_ref[...], axis=1, keepdims=True) + ba_ref[0, 0])
  eta = jax.nn.sigmoid(
      jnp.sum(i_norm * we_ref[...], axis=1, keepdims=True) + be_ref[0, 0]
  ) * (D ** -0.5)
  krep = _dot(k, rep)
  pred = _dot(old * krep, seg)
  err = v - pred
  errrep = _dot(err, expm)
  new_ref[...] = (1.0 - alpha) * old + eta * (errrep * krep)


def _tc_dense(use, item, old2, rep, expm, seg, Wq, Wk, Wv, Wout, bout,
              w_alpha, b_alpha, w_eta, b_eta, W1, b1, W2, b2,
              ln1_g, ln1_b, ln2_g, ln2_b):
  full = lambda s: pl.BlockSpec(s, lambda i: tuple(0 for _ in s))
  return pl.pallas_call(
      _tc_body,
      grid=(GRID,),
      in_specs=[
          pl.BlockSpec((BSZ, D), lambda i: (i, 0)),
          pl.BlockSpec((BSZ, D), lambda i: (i, 0)),
          pl.BlockSpec((BSZ, DD), lambda i: (i, 0)),
          full((D, DD)), full((D, DD)), full((DD, D)),
          full((D, D)), full((D, D)), full((D, D)), full((D, D)),
          full((1, D)),
          full((1, D)), full((1, 1)),
          full((1, D)), full((1, 1)),
          full((DFF, D)), full((1, DFF)),
          full((D, DFF)), full((1, D)),
          full((1, D)), full((1, D)), full((1, D)), full((1, D)),
      ],
      out_specs=[
          pl.BlockSpec((BSZ, D), lambda i: (i, 0)),
          pl.BlockSpec((BSZ, DD), lambda i: (i, 0)),
      ],
      out_shape=[
          jax.ShapeDtypeStruct((B, D), jnp.float32),
          jax.ShapeDtypeStruct((B, DD), jnp.float32),
      ],
      compiler_params=pltpu.CompilerParams(
          dimension_semantics=("arbitrary",)),
  )(use, item, old2, rep, expm, seg, Wq, Wk, Wv, Wout, bout,
    w_alpha, b_alpha, w_eta, b_eta, W1, b1, W2, b2,
    ln1_g, ln1_b, ln2_g, ln2_b)


def kernel(user_ids, user_static_emb, item_emb, states, Wq, Wk, Wv, Wout, bout,
           w_alpha, b_alpha, w_eta, b_eta, W1, b1, W2, b2,
           ln1_g, ln1_b, ln2_g, ln2_b):
  ids = user_ids.astype(jnp.int32)
  st_ref = jax.new_ref(states.reshape(M, DD))

  old_flat = _sc_gather_ref(st_ref, ids)

  win3 = _win_idx(ids.reshape(1, B), ids.reshape(B, 1))

  pcol = lax.broadcasted_iota(jnp.int32, (D, DD), 1)
  prow = lax.broadcasted_iota(jnp.int32, (D, DD), 0)
  rep = (pcol % D == prow).astype(jnp.float32)
  expm = (pcol // D == prow).astype(jnp.float32)
  seg = expm.T

  ude, new2 = _tc_dense(
      user_static_emb, item_emb, old_flat, rep, expm, seg,
      Wq, Wk, Wv, Wout, bout.reshape(1, D),
      w_alpha, b_alpha.reshape(1, 1), w_eta, b_eta.reshape(1, 1),
      W1, b1.reshape(1, DFF), W2, b2.reshape(1, D),
      ln1_g.reshape(1, D), ln1_b.reshape(1, D),
      ln2_g.reshape(1, D), ln2_b.reshape(1, D))

  _sc_scatter(new2, ids, win3.reshape(B), st_ref)
  return ude, st_ref[...].reshape(M, D, D)


# speedup vs baseline: 2.7458x; 1.0500x over previous
import functools

import jax
import jax.numpy as jnp
from jax import lax
from jax.experimental import pallas as pl
from jax.experimental.pallas import tpu as pltpu
from jax.experimental.pallas import tpu_sc as plsc

B = 4096
D = 64
DFF = 2048
M = 50000
DD = D * D

NC = 2
NS = 16
NW = NC * NS
L = 16
BPW = B // NW
GROUPS = BPW // L

_sc_mesh = plsc.VectorSubcoreMesh(core_axis_name="c", subcore_axis_name="s")


@functools.partial(
    pl.kernel,
    out_type=jax.ShapeDtypeStruct((B, DD), jnp.float32),
    mesh=_sc_mesh,
    scratch_types=[
        pltpu.VMEM((BPW,), jnp.int32),
        pltpu.VMEM((L, DD), jnp.float32),
        pltpu.SemaphoreType.DMA,
    ],
)
def _sc_gather(states_hbm, ids_hbm, out_hbm, idx_v, rows_v, sem):
  wid = lax.axis_index("s") * NC + lax.axis_index("c")
  base = wid * BPW
  pltpu.sync_copy(ids_hbm.at[pl.ds(base, BPW)], idx_v)
  for g in range(GROUPS):
    idxs = idx_v[pl.ds(g * L, L)]
    pltpu.async_copy(states_hbm.at[idxs], rows_v, sem).wait()
    pltpu.sync_copy(rows_v, out_hbm.at[pl.ds(base + g * L, L)])


@functools.partial(
    pl.kernel,
    out_type=(),
    mesh=_sc_mesh,
    scratch_types=[
        pltpu.VMEM((BPW,), jnp.int32),
        pltpu.VMEM((BPW,), jnp.int32),
        pltpu.VMEM((L, DD), jnp.float32),
        pltpu.SemaphoreType.DMA,
    ],
)
def _sc_scatter(new_hbm, ids_hbm, win_hbm, states_ref, idx_v, win_v, rows_v, sem):
  wid = lax.axis_index("s") * NC + lax.axis_index("c")
  base = wid * BPW
  pltpu.sync_copy(ids_hbm.at[pl.ds(base, BPW)], idx_v)
  pltpu.sync_copy(win_hbm.at[pl.ds(base, BPW)], win_v)
  for g in range(GROUPS):
    wins = win_v[pl.ds(g * L, L)]
    pltpu.async_copy(new_hbm.at[wins], rows_v, sem).wait()
    dsts = idx_v[pl.ds(g * L, L)]
    pltpu.async_copy(rows_v, states_ref.at[dsts], sem).wait()


_sc_gather_ref = _sc_gather


TCOLS = 128
TGRID = (M + TCOLS - 1) // TCOLS


def _tp_body(src_ref, dst_ref):
  dst_ref[...] = src_ref[...].T


def _tc_transpose(st_t):
  return pl.pallas_call(
      _tp_body,
      grid=(TGRID,),
      in_specs=[pl.BlockSpec((DD, TCOLS), lambda i: (0, i))],
      out_specs=pl.BlockSpec((TCOLS, DD), lambda i: (i, 0)),
      out_shape=jax.ShapeDtypeStruct((M, DD), jnp.float32),
      compiler_params=pltpu.CompilerParams(
          dimension_semantics=("arbitrary",)),
  )(st_t)


WSZ = 256
WGRID = B // WSZ


def _win_body(ids_row_ref, ids_col_ref, win_ref):
  eq = ids_col_ref[...] == ids_row_ref[...]
  pos = lax.broadcasted_iota(jnp.int32, (WSZ, B), 1)
  win_ref[0, 0, :] = jnp.max(jnp.where(eq, pos, -1), axis=1)


def _win_idx(ids_row, ids_col):
  return pl.pallas_call(
      _win_body,
      grid=(WGRID,),
      in_specs=[
          pl.BlockSpec((1, B), lambda i: (0, 0)),
          pl.BlockSpec((WSZ, 1), lambda i: (i, 0)),
      ],
      out_specs=pl.BlockSpec((1, 1, WSZ), lambda i: (i, 0, 0)),
      out_shape=jax.ShapeDtypeStruct((WGRID, 1, WSZ), jnp.int32),
      compiler_params=pltpu.CompilerParams(
          dimension_semantics=("arbitrary",)),
  )(ids_row, ids_col)


BSZ = 128
GRID = B // BSZ


def _ln(x, g, b):
  mu = jnp.mean(x, axis=-1, keepdims=True)
  var = jnp.mean((x - mu) * (x - mu), axis=-1, keepdims=True)
  return (x - mu) * lax.rsqrt(var + 1e-5) * g + b


def _l2norm(x):
  n = jnp.sqrt(jnp.sum(x * x, axis=-1, keepdims=True))
  return x / jnp.maximum(n, 1e-12)


def _dot_t(a, b):
  return lax.dot_general(a, b, (((1,), (1,)), ((), ())),
                         preferred_element_type=jnp.float32)


def _dot(a, b):
  return lax.dot_general(a, b, (((1,), (0,)), ((), ())),
                         preferred_element_type=jnp.float32)


def _tc_body(use_ref, item_ref, old_ref, rep_ref, exp_ref, seg_ref,
             Wq_ref, Wk_ref, Wv_ref, Wo_ref, bo_ref, wa_ref, ba_ref,
             we_ref, be_ref, W1_ref, b1_ref, W2_ref, b2_ref,
             g1_ref, c1_ref, g2_ref, c2_ref,
             ude_ref, new_ref):
  use = use_ref[...]
  item = item_ref[...]
  old = old_ref[...]
  rep = rep_ref[...]
  expm = exp_ref[...]
  seg = seg_ref[...]
  g1, c1 = g1_ref[...], c1_ref[...]
  g2, c2 = g2_ref[...], c2_ref[...]

  x_norm = _ln(use, g1, c1)
  q = _l2norm(_dot_t(x_norm, Wq_ref[...]))
  qrep = _dot(q, rep)
  read_content = _dot(old * qrep, seg)
  attn = _dot_t(read_content, Wo_ref[...]) + bo_ref[...]
  x = use + attn
  x2 = _ln(x, g2, c2)
  h = jnp.maximum(_dot_t(x2, W1_ref[...]) + b1_ref[...], 0.0)
  ffn = _dot_t(h, W2_ref[...]) + b2_ref[...]
  ude_ref[...] = x + ffn

  i_norm = _ln(item, g1, c1)
  k = _l2norm(_dot_t(i_norm, Wk_ref[...]))
  v = _dot_t(i_norm, Wv_ref[...])
  alpha = jax.nn.sigmoid(
      jnp.sum(i_norm * wa_ref[...], axis=1, keepdims=True) + ba_ref[0, 0])
  eta = jax.nn.sigmoid(
      jnp.sum(i_norm * we_ref[...], axis=1, keepdims=True) + be_ref[0, 0]
  ) * (D ** -0.5)
  krep = _dot(k, rep)
  pred = _dot(old * krep, seg)
  err = v - pred
  errrep = _dot(err, expm)
  new_ref[...] = (1.0 - alpha) * old + eta * (errrep * krep)


def _tc_dense(use, item, old2, rep, expm, seg, Wq, Wk, Wv, Wout, bout,
              w_alpha, b_alpha, w_eta, b_eta, W1, b1, W2, b2,
              ln1_g, ln1_b, ln2_g, ln2_b):
  full = lambda s: pl.BlockSpec(s, lambda i: tuple(0 for _ in s))
  return pl.pallas_call(
      _tc_body,
      grid=(GRID,),
      in_specs=[
          pl.BlockSpec((BSZ, D), lambda i: (i, 0)),
          pl.BlockSpec((BSZ, D), lambda i: (i, 0)),
          pl.BlockSpec((BSZ, DD), lambda i: (i, 0)),
          full((D, DD)), full((D, DD)), full((DD, D)),
          full((D, D)), full((D, D)), full((D, D)), full((D, D)),
          full((1, D)),
          full((1, D)), full((1, 1)),
          full((1, D)), full((1, 1)),
          full((DFF, D)), full((1, DFF)),
          full((D, DFF)), full((1, D)),
          full((1, D)), full((1, D)), full((1, D)), full((1, D)),
      ],
      out_specs=[
          pl.BlockSpec((BSZ, D), lambda i: (i, 0)),
          pl.BlockSpec((BSZ, DD), lambda i: (i, 0)),
      ],
      out_shape=[
          jax.ShapeDtypeStruct((B, D), jnp.float32),
          jax.ShapeDtypeStruct((B, DD), jnp.float32),
      ],
      compiler_params=pltpu.CompilerParams(
          dimension_semantics=("arbitrary",)),
  )(use, item, old2, rep, expm, seg, Wq, Wk, Wv, Wout, bout,
    w_alpha, b_alpha, w_eta, b_eta, W1, b1, W2, b2,
    ln1_g, ln1_b, ln2_g, ln2_b)


def kernel(user_ids, user_static_emb, item_emb, states, Wq, Wk, Wv, Wout, bout,
           w_alpha, b_alpha, w_eta, b_eta, W1, b1, W2, b2,
           ln1_g, ln1_b, ln2_g, ln2_b):
  ids = user_ids.astype(jnp.int32)
  st_t = states.transpose(1, 2, 0).reshape(DD, M)
  st_ref = jax.new_ref(_tc_transpose(st_t))

  old_flat = _sc_gather_ref(st_ref, ids)

  win3 = _win_idx(ids.reshape(1, B), ids.reshape(B, 1))

  pcol = lax.broadcasted_iota(jnp.int32, (D, DD), 1)
  prow = lax.broadcasted_iota(jnp.int32, (D, DD), 0)
  rep = (pcol % D == prow).astype(jnp.float32)
  expm = (pcol // D == prow).astype(jnp.float32)
  seg = expm.T

  ude, new2 = _tc_dense(
      user_static_emb, item_emb, old_flat, rep, expm, seg,
      Wq, Wk, Wv, Wout, bout.reshape(1, D),
      w_alpha, b_alpha.reshape(1, 1), w_eta, b_eta.reshape(1, 1),
      W1, b1.reshape(1, DFF), W2, b2.reshape(1, D),
      ln1_g.reshape(1, D), ln1_b.reshape(1, D),
      ln2_g.reshape(1, D), ln2_b.reshape(1, D))

  _sc_scatter(new2, ids, win3.reshape(B), st_ref)
  return ude, st_ref[...].reshape(M, D, D)


# speedup vs baseline: 2.9015x; 1.0567x over previous
import functools

import jax
import jax.numpy as jnp
from jax import lax
from jax.experimental import pallas as pl
from jax.experimental.pallas import tpu as pltpu
from jax.experimental.pallas import tpu_sc as plsc

B = 4096
D = 64
DFF = 2048
M = 50000
DD = D * D

NC = 2
NS = 16
NW = NC * NS
L = 16
BPW = B // NW
GROUPS = BPW // L

_sc_mesh = plsc.VectorSubcoreMesh(core_axis_name="c", subcore_axis_name="s")


@functools.partial(
    pl.kernel,
    out_type=jax.ShapeDtypeStruct((B, DD), jnp.float32),
    mesh=_sc_mesh,
    scratch_types=[
        pltpu.VMEM((BPW,), jnp.int32),
        pltpu.VMEM((L, DD), jnp.float32),
        pltpu.SemaphoreType.DMA,
    ],
)
def _sc_gather(states_hbm, ids_hbm, out_hbm, idx_v, rows_v, sem):
  wid = lax.axis_index("s") * NC + lax.axis_index("c")
  base = wid * BPW
  pltpu.sync_copy(ids_hbm.at[pl.ds(base, BPW)], idx_v)
  for g in range(GROUPS):
    idxs = idx_v[pl.ds(g * L, L)]
    pltpu.async_copy(states_hbm.at[idxs], rows_v, sem).wait()
    pltpu.sync_copy(rows_v, out_hbm.at[pl.ds(base + g * L, L)])


@functools.partial(
    pl.kernel,
    out_type=(),
    mesh=_sc_mesh,
    scratch_types=[
        pltpu.VMEM((BPW,), jnp.int32),
        pltpu.VMEM((BPW,), jnp.int32),
        pltpu.VMEM((L, DD), jnp.float32),
        pltpu.SemaphoreType.DMA,
    ],
)
def _sc_scatter(new_hbm, ids_hbm, win_hbm, states_ref, idx_v, win_v, rows_v, sem):
  wid = lax.axis_index("s") * NC + lax.axis_index("c")
  base = wid * BPW
  pltpu.sync_copy(ids_hbm.at[pl.ds(base, BPW)], idx_v)
  pltpu.sync_copy(win_hbm.at[pl.ds(base, BPW)], win_v)
  for g in range(GROUPS):
    wins = win_v[pl.ds(g * L, L)]
    pltpu.async_copy(new_hbm.at[wins], rows_v, sem).wait()
    dsts = idx_v[pl.ds(g * L, L)]
    pltpu.async_copy(rows_v, states_ref.at[dsts], sem).wait()


_sc_gather_ref = _sc_gather


TCOLS = 128
TGRID = (M + TCOLS - 1) // TCOLS


def _tp_body(src_ref, dst_ref):
  dst_ref[...] = src_ref[...].T


def _tc_transpose(st_t):
  return pl.pallas_call(
      _tp_body,
      grid=(TGRID,),
      in_specs=[pl.BlockSpec((DD, TCOLS), lambda i: (0, i))],
      out_specs=pl.BlockSpec((TCOLS, DD), lambda i: (i, 0)),
      out_shape=jax.ShapeDtypeStruct((M, DD), jnp.float32),
      compiler_params=pltpu.CompilerParams(
          dimension_semantics=("arbitrary",)),
  )(st_t)


def _tpo_body(src_ref, dst_ref):
  dst_ref[...] = src_ref[...].T


def _tc_transpose_out(st_rm):
  return pl.pallas_call(
      _tpo_body,
      grid=(TGRID,),
      in_specs=[pl.BlockSpec((TCOLS, DD), lambda i: (i, 0))],
      out_specs=pl.BlockSpec((DD, TCOLS), lambda i: (0, i)),
      out_shape=jax.ShapeDtypeStruct((DD, M), jnp.float32),
      compiler_params=pltpu.CompilerParams(
          dimension_semantics=("arbitrary",)),
  )(st_rm)


WSZ = 256
WGRID = B // WSZ


def _win_body(ids_row_ref, ids_col_ref, win_ref):
  eq = ids_col_ref[...] == ids_row_ref[...]
  pos = lax.broadcasted_iota(jnp.int32, (WSZ, B), 1)
  win_ref[0, 0, :] = jnp.max(jnp.where(eq, pos, -1), axis=1)


def _win_idx(ids_row, ids_col):
  return pl.pallas_call(
      _win_body,
      grid=(WGRID,),
      in_specs=[
          pl.BlockSpec((1, B), lambda i: (0, 0)),
          pl.BlockSpec((WSZ, 1), lambda i: (i, 0)),
      ],
      out_specs=pl.BlockSpec((1, 1, WSZ), lambda i: (i, 0, 0)),
      out_shape=jax.ShapeDtypeStruct((WGRID, 1, WSZ), jnp.int32),
      compiler_params=pltpu.CompilerParams(
          dimension_semantics=("arbitrary",)),
  )(ids_row, ids_col)


BSZ = 128
GRID = B // BSZ


def _ln(x, g, b):
  mu = jnp.mean(x, axis=-1, keepdims=True)
  var = jnp.mean((x - mu) * (x - mu), axis=-1, keepdims=True)
  return (x - mu) * lax.rsqrt(var + 1e-5) * g + b


def _l2norm(x):
  n = jnp.sqrt(jnp.sum(x * x, axis=-1, keepdims=True))
  return x / jnp.maximum(n, 1e-12)


def _dot_t(a, b):
  return lax.dot_general(a, b, (((1,), (1,)), ((), ())),
                         preferred_element_type=jnp.float32)


def _dot(a, b):
  return lax.dot_general(a, b, (((1,), (0,)), ((), ())),
                         preferred_element_type=jnp.float32)


def _tc_body(use_ref, item_ref, old_ref, rep_ref, exp_ref, seg_ref,
             Wq_ref, Wk_ref, Wv_ref, Wo_ref, bo_ref, wa_ref, ba_ref,
             we_ref, be_ref, W1_ref, b1_ref, W2_ref, b2_ref,
             g1_ref, c1_ref, g2_ref, c2_ref,
             ude_ref, new_ref):
  use = use_ref[...]
  item = item_ref[...]
  old = old_ref[...]
  rep = rep_ref[...]
  expm = exp_ref[...]
  seg = seg_ref[...]
  g1, c1 = g1_ref[...], c1_ref[...]
  g2, c2 = g2_ref[...], c2_ref[...]

  x_norm = _ln(use, g1, c1)
  q = _l2norm(_dot_t(x_norm, Wq_ref[...]))
  qrep = _dot(q, rep)
  read_content = _dot(old * qrep, seg)
  attn = _dot_t(read_content, Wo_ref[...]) + bo_ref[...]
  x = use + attn
  x2 = _ln(x, g2, c2)
  h = jnp.maximum(_dot_t(x2, W1_ref[...]) + b1_ref[...], 0.0)
  ffn = _dot_t(h, W2_ref[...]) + b2_ref[...]
  ude_ref[...] = x + ffn

  i_norm = _ln(item, g1, c1)
  k = _l2norm(_dot_t(i_norm, Wk_ref[...]))
  v = _dot_t(i_norm, Wv_ref[...])
  alpha = jax.nn.sigmoid(
      jnp.sum(i_norm * wa_ref[...], axis=1, keepdims=True) + ba_ref[0, 0])
  eta = jax.nn.sigmoid(
      jnp.sum(i_norm * we_ref[...], axis=1, keepdims=True) + be_ref[0, 0]
  ) * (D ** -0.5)
  krep = _dot(k, rep)
  pred = _dot(old * krep, seg)
  err = v - pred
  errrep = _dot(err, expm)
  new_ref[...] = (1.0 - alpha) * old + eta * (errrep * krep)


def _tc_dense(use, item, old2, rep, expm, seg, Wq, Wk, Wv, Wout, bout,
              w_alpha, b_alpha, w_eta, b_eta, W1, b1, W2, b2,
              ln1_g, ln1_b, ln2_g, ln2_b):
  full = lambda s: pl.BlockSpec(s, lambda i: tuple(0 for _ in s))
  return pl.pallas_call(
      _tc_body,
      grid=(GRID,),
      in_specs=[
          pl.BlockSpec((BSZ, D), lambda i: (i, 0)),
          pl.BlockSpec((BSZ, D), lambda i: (i, 0)),
          pl.BlockSpec((BSZ, DD), lambda i: (i, 0)),
          full((D, DD)), full((D, DD)), full((DD, D)),
          full((D, D)), full((D, D)), full((D, D)), full((D, D)),
          full((1, D)),
          full((1, D)), full((1, 1)),
          full((1, D)), full((1, 1)),
          full((DFF, D)), full((1, DFF)),
          full((D, DFF)), full((1, D)),
          full((1, D)), full((1, D)), full((1, D)), full((1, D)),
      ],
      out_specs=[
          pl.BlockSpec((BSZ, D), lambda i: (i, 0)),
          pl.BlockSpec((BSZ, DD), lambda i: (i, 0)),
      ],
      out_shape=[
          jax.ShapeDtypeStruct((B, D), jnp.float32),
          jax.ShapeDtypeStruct((B, DD), jnp.float32),
      ],
      compiler_params=pltpu.CompilerParams(
          dimension_semantics=("arbitrary",)),
  )(use, item, old2, rep, expm, seg, Wq, Wk, Wv, Wout, bout,
    w_alpha, b_alpha, w_eta, b_eta, W1, b1, W2, b2,
    ln1_g, ln1_b, ln2_g, ln2_b)


def kernel(user_ids, user_static_emb, item_emb, states, Wq, Wk, Wv, Wout, bout,
           w_alpha, b_alpha, w_eta, b_eta, W1, b1, W2, b2,
           ln1_g, ln1_b, ln2_g, ln2_b):
  ids = user_ids.astype(jnp.int32)
  st_t = states.transpose(1, 2, 0).reshape(DD, M)
  st_ref = jax.new_ref(_tc_transpose(st_t))

  old_flat = _sc_gather_ref(st_ref, ids)

  win3 = _win_idx(ids.reshape(1, B), ids.reshape(B, 1))

  pcol = lax.broadcasted_iota(jnp.int32, (D, DD), 1)
  prow = lax.broadcasted_iota(jnp.int32, (D, DD), 0)
  rep = (pcol % D == prow).astype(jnp.float32)
  expm = (pcol // D == prow).astype(jnp.float32)
  seg = expm.T

  ude, new2 = _tc_dense(
      user_static_emb, item_emb, old_flat, rep, expm, seg,
      Wq, Wk, Wv, Wout, bout.reshape(1, D),
      w_alpha, b_alpha.reshape(1, 1), w_eta, b_eta.reshape(1, 1),
      W1, b1.reshape(1, DFF), W2, b2.reshape(1, D),
      ln1_g.reshape(1, D), ln1_b.reshape(1, D),
      ln2_g.reshape(1, D), ln2_b.reshape(1, D))

  _sc_scatter(new2, ids, win3.reshape(B), st_ref)
  out_t = _tc_transpose_out(st_ref[...])
  return ude, out_t.reshape(D, D, M).transpose(2, 0, 1)


# speedup vs baseline: 3.3374x; 1.1502x over previous
import functools

import jax
import jax.numpy as jnp
from jax import lax
from jax.experimental import pallas as pl
from jax.experimental.pallas import tpu as pltpu
from jax.experimental.pallas import tpu_sc as plsc

B = 4096
D = 64
DFF = 2048
M = 50000
DD = D * D

NC = 2
NS = 16
NW = NC * NS
L = 16
BPW = B // NW
GROUPS = BPW // L

_sc_mesh = plsc.VectorSubcoreMesh(core_axis_name="c", subcore_axis_name="s")


@functools.partial(
    pl.kernel,
    out_type=jax.ShapeDtypeStruct((B, DD), jnp.float32),
    mesh=_sc_mesh,
    scratch_types=[
        pltpu.VMEM((BPW,), jnp.int32),
        pltpu.VMEM((L, DD), jnp.float32),
        pltpu.SemaphoreType.DMA,
    ],
)
def _sc_gather(states_hbm, ids_hbm, out_hbm, idx_v, rows_v, sem):
  wid = lax.axis_index("s") * NC + lax.axis_index("c")
  base = wid * BPW
  pltpu.sync_copy(ids_hbm.at[pl.ds(base, BPW)], idx_v)
  for g in range(GROUPS):
    idxs = idx_v[pl.ds(g * L, L)]
    pltpu.async_copy(states_hbm.at[idxs], rows_v, sem).wait()
    pltpu.sync_copy(rows_v, out_hbm.at[pl.ds(base + g * L, L)])


@functools.partial(
    pl.kernel,
    out_type=(),
    mesh=_sc_mesh,
    scratch_types=[
        pltpu.VMEM((BPW,), jnp.int32),
        pltpu.VMEM((BPW,), jnp.int32),
        pltpu.VMEM((L, DD), jnp.float32),
        pltpu.SemaphoreType.DMA,
    ],
)
def _sc_scatter(new_hbm, ids_hbm, win_hbm, states_ref, idx_v, win_v, rows_v, sem):
  wid = lax.axis_index("s") * NC + lax.axis_index("c")
  base = wid * BPW
  pltpu.sync_copy(ids_hbm.at[pl.ds(base, BPW)], idx_v)
  pltpu.sync_copy(win_hbm.at[pl.ds(base, BPW)], win_v)
  for g in range(GROUPS):
    wins = win_v[pl.ds(g * L, L)]
    pltpu.async_copy(new_hbm.at[wins], rows_v, sem).wait()
    dsts = idx_v[pl.ds(g * L, L)]
    pltpu.async_copy(rows_v, states_ref.at[dsts], sem).wait()


_sc_gather_ref = _sc_gather


TCOLS = 256
TGRID = (M + TCOLS - 1) // TCOLS


def _tp_body(src_ref, dst_ref):
  dst_ref[...] = src_ref[...].T


def _tc_transpose(st_t):
  return pl.pallas_call(
      _tp_body,
      grid=(TGRID,),
      in_specs=[pl.BlockSpec((DD, TCOLS), lambda i: (0, i))],
      out_specs=pl.BlockSpec((TCOLS, DD), lambda i: (i, 0)),
      out_shape=jax.ShapeDtypeStruct((M, DD), jnp.float32),
      compiler_params=pltpu.CompilerParams(
          dimension_semantics=("arbitrary",)),
  )(st_t)


def _tpo_body(src_ref, dst_ref):
  dst_ref[...] = src_ref[...].T


def _tc_transpose_out(st_rm):
  return pl.pallas_call(
      _tpo_body,
      grid=(TGRID,),
      in_specs=[pl.BlockSpec((TCOLS, DD), lambda i: (i, 0))],
      out_specs=pl.BlockSpec((DD, TCOLS), lambda i: (0, i)),
      out_shape=jax.ShapeDtypeStruct((DD, M), jnp.float32),
      compiler_params=pltpu.CompilerParams(
          dimension_semantics=("arbitrary",)),
  )(st_rm)


WSZ = 256
WGRID = B // WSZ


def _win_body(ids_row_ref, ids_col_ref, win_ref):
  eq = ids_col_ref[...] == ids_row_ref[...]
  pos = lax.broadcasted_iota(jnp.int32, (WSZ, B), 1)
  win_ref[0, 0, :] = jnp.max(jnp.where(eq, pos, -1), axis=1)


def _win_idx(ids_row, ids_col):
  return pl.pallas_call(
      _win_body,
      grid=(WGRID,),
      in_specs=[
          pl.BlockSpec((1, B), lambda i: (0, 0)),
          pl.BlockSpec((WSZ, 1), lambda i: (i, 0)),
      ],
      out_specs=pl.BlockSpec((1, 1, WSZ), lambda i: (i, 0, 0)),
      out_shape=jax.ShapeDtypeStruct((WGRID, 1, WSZ), jnp.int32),
      compiler_params=pltpu.CompilerParams(
          dimension_semantics=("arbitrary",)),
  )(ids_row, ids_col)


BSZ = 128
GRID = B // BSZ


def _ln(x, g, b):
  mu = jnp.mean(x, axis=-1, keepdims=True)
  var = jnp.mean((x - mu) * (x - mu), axis=-1, keepdims=True)
  return (x - mu) * lax.rsqrt(var + 1e-5) * g + b


def _l2norm(x):
  n = jnp.sqrt(jnp.sum(x * x, axis=-1, keepdims=True))
  return x / jnp.maximum(n, 1e-12)


def _dot_t(a, b):
  return lax.dot_general(a, b, (((1,), (1,)), ((), ())),
                         preferred_element_type=jnp.float32)


def _dot(a, b):
  return lax.dot_general(a, b, (((1,), (0,)), ((), ())),
                         preferred_element_type=jnp.float32)


def _tc_body(use_ref, item_ref, old_ref, rep_ref, exp_ref, seg_ref,
             Wq_ref, Wk_ref, Wv_ref, Wo_ref, bo_ref, wa_ref, ba_ref,
             we_ref, be_ref, W1_ref, b1_ref, W2_ref, b2_ref,
             g1_ref, c1_ref, g2_ref, c2_ref,
             ude_ref, new_ref):
  use = use_ref[...]
  item = item_ref[...]
  old = old_ref[...]
  rep = rep_ref[...]
  expm = exp_ref[...]
  seg = seg_ref[...]
  g1, c1 = g1_ref[...], c1_ref[...]
  g2, c2 = g2_ref[...], c2_ref[...]

  x_norm = _ln(use, g1, c1)
  q = _l2norm(_dot_t(x_norm, Wq_ref[...]))
  qrep = _dot(q, rep)
  read_content = _dot(old * qrep, seg)
  attn = _dot_t(read_content, Wo_ref[...]) + bo_ref[...]
  x = use + attn
  x2 = _ln(x, g2, c2)
  h = jnp.maximum(_dot_t(x2, W1_ref[...]) + b1_ref[...], 0.0)
  ffn = _dot_t(h, W2_ref[...]) + b2_ref[...]
  ude_ref[...] = x + ffn

  i_norm = _ln(item, g1, c1)
  k = _l2norm(_dot_t(i_norm, Wk_ref[...]))
  v = _dot_t(i_norm, Wv_ref[...])
  alpha = jax.nn.sigmoid(
      jnp.sum(i_norm * wa_ref[...], axis=1, keepdims=True) + ba_ref[0, 0])
  eta = jax.nn.sigmoid(
      jnp.sum(i_norm * we_ref[...], axis=1, keepdims=True) + be_ref[0, 0]
  ) * (D ** -0.5)
  krep = _dot(k, rep)
  pred = _dot(old * krep, seg)
  err = v - pred
  errrep = _dot(err, expm)
  new_ref[...] = (1.0 - alpha) * old + eta * (errrep * krep)


def _tc_dense(use, item, old2, rep, expm, seg, Wq, Wk, Wv, Wout, bout,
              w_alpha, b_alpha, w_eta, b_eta, W1, b1, W2, b2,
              ln1_g, ln1_b, ln2_g, ln2_b):
  full = lambda s: pl.BlockSpec(s, lambda i: tuple(0 for _ in s))
  return pl.pallas_call(
      _tc_body,
      grid=(GRID,),
      in_specs=[
          pl.BlockSpec((BSZ, D), lambda i: (i, 0)),
          pl.BlockSpec((BSZ, D), lambda i: (i, 0)),
          pl.BlockSpec((BSZ, DD), lambda i: (i, 0)),
          full((D, DD)), full((D, DD)), full((DD, D)),
          full((D, D)), full((D, D)), full((D, D)), full((D, D)),
          full((1, D)),
          full((1, D)), full((1, 1)),
          full((1, D)), full((1, 1)),
          full((DFF, D)), full((1, DFF)),
          full((D, DFF)), full((1, D)),
          full((1, D)), full((1, D)), full((1, D)), full((1, D)),
      ],
      out_specs=[
          pl.BlockSpec((BSZ, D), lambda i: (i, 0)),
          pl.BlockSpec((BSZ, DD), lambda i: (i, 0)),
      ],
      out_shape=[
          jax.ShapeDtypeStruct((B, D), jnp.float32),
          jax.ShapeDtypeStruct((B, DD), jnp.float32),
      ],
      compiler_params=pltpu.CompilerParams(
          dimension_semantics=("arbitrary",)),
  )(use, item, old2, rep, expm, seg, Wq, Wk, Wv, Wout, bout,
    w_alpha, b_alpha, w_eta, b_eta, W1, b1, W2, b2,
    ln1_g, ln1_b, ln2_g, ln2_b)


def kernel(user_ids, user_static_emb, item_emb, states, Wq, Wk, Wv, Wout, bout,
           w_alpha, b_alpha, w_eta, b_eta, W1, b1, W2, b2,
           ln1_g, ln1_b, ln2_g, ln2_b):
  ids = user_ids.astype(jnp.int32)
  st_t = states.transpose(1, 2, 0).reshape(DD, M)
  st_ref = jax.new_ref(_tc_transpose(st_t))

  old_flat = _sc_gather_ref(st_ref, ids)

  win3 = _win_idx(ids.reshape(1, B), ids.reshape(B, 1))

  pcol = lax.broadcasted_iota(jnp.int32, (D, DD), 1)
  prow = lax.broadcasted_iota(jnp.int32, (D, DD), 0)
  rep = (pcol % D == prow).astype(jnp.float32)
  expm = (pcol // D == prow).astype(jnp.float32)
  seg = expm.T

  ude, new2 = _tc_dense(
      user_static_emb, item_emb, old_flat, rep, expm, seg,
      Wq, Wk, Wv, Wout, bout.reshape(1, D),
      w_alpha, b_alpha.reshape(1, 1), w_eta, b_eta.reshape(1, 1),
      W1, b1.reshape(1, DFF), W2, b2.reshape(1, D),
      ln1_g.reshape(1, D), ln1_b.reshape(1, D),
      ln2_g.reshape(1, D), ln2_b.reshape(1, D))

  _sc_scatter(new2, ids, win3.reshape(B), st_ref)
  out_t = _tc_transpose_out(st_ref[...])
  return ude, out_t.reshape(D, D, M).transpose(2, 0, 1)


# speedup vs baseline: 3.4208x; 1.0250x over previous
import functools

import jax
import jax.numpy as jnp
from jax import lax
from jax.experimental import pallas as pl
from jax.experimental.pallas import tpu as pltpu
from jax.experimental.pallas import tpu_sc as plsc

B = 4096
D = 64
DFF = 2048
M = 50000
DD = D * D

NC = 2
NS = 16
NW = NC * NS
L = 16
BPW = B // NW
GROUPS = BPW // L

_sc_mesh = plsc.VectorSubcoreMesh(core_axis_name="c", subcore_axis_name="s")


@functools.partial(
    pl.kernel,
    out_type=jax.ShapeDtypeStruct((B, DD), jnp.float32),
    mesh=_sc_mesh,
    scratch_types=[
        pltpu.VMEM((BPW,), jnp.int32),
        pltpu.VMEM((L, DD), jnp.float32),
        pltpu.SemaphoreType.DMA,
    ],
)
def _sc_gather(states_hbm, ids_hbm, out_hbm, idx_v, rows_v, sem):
  wid = lax.axis_index("s") * NC + lax.axis_index("c")
  base = wid * BPW
  pltpu.sync_copy(ids_hbm.at[pl.ds(base, BPW)], idx_v)
  for g in range(GROUPS):
    idxs = idx_v[pl.ds(g * L, L)]
    pltpu.async_copy(states_hbm.at[idxs], rows_v, sem).wait()
    pltpu.sync_copy(rows_v, out_hbm.at[pl.ds(base + g * L, L)])


@functools.partial(
    pl.kernel,
    out_type=(),
    mesh=_sc_mesh,
    scratch_types=[
        pltpu.VMEM((BPW,), jnp.int32),
        pltpu.VMEM((BPW,), jnp.int32),
        pltpu.VMEM((L, DD), jnp.float32),
        pltpu.SemaphoreType.DMA,
    ],
)
def _sc_scatter(new_hbm, ids_hbm, win_hbm, states_ref, idx_v, win_v, rows_v, sem):
  wid = lax.axis_index("s") * NC + lax.axis_index("c")
  base = wid * BPW
  pltpu.sync_copy(ids_hbm.at[pl.ds(base, BPW)], idx_v)
  pltpu.sync_copy(win_hbm.at[pl.ds(base, BPW)], win_v)
  for g in range(GROUPS):
    wins = win_v[pl.ds(g * L, L)]
    pltpu.async_copy(new_hbm.at[wins], rows_v, sem).wait()
    dsts = idx_v[pl.ds(g * L, L)]
    pltpu.async_copy(rows_v, states_ref.at[dsts], sem).wait()


_sc_gather_ref = _sc_gather


TCOLS = 512
TGRID = (M + TCOLS - 1) // TCOLS


def _tp_body(src_ref, dst_ref):
  dst_ref[...] = src_ref[...].T


def _tc_transpose(st_t):
  return pl.pallas_call(
      _tp_body,
      grid=(TGRID,),
      in_specs=[pl.BlockSpec((DD, TCOLS), lambda i: (0, i))],
      out_specs=pl.BlockSpec((TCOLS, DD), lambda i: (i, 0)),
      out_shape=jax.ShapeDtypeStruct((M, DD), jnp.float32),
      compiler_params=pltpu.CompilerParams(
          dimension_semantics=("arbitrary",)),
  )(st_t)


def _tpo_body(src_ref, dst_ref):
  dst_ref[...] = src_ref[...].T


def _tc_transpose_out(st_rm):
  return pl.pallas_call(
      _tpo_body,
      grid=(TGRID,),
      in_specs=[pl.BlockSpec((TCOLS, DD), lambda i: (i, 0))],
      out_specs=pl.BlockSpec((DD, TCOLS), lambda i: (0, i)),
      out_shape=jax.ShapeDtypeStruct((DD, M), jnp.float32),
      compiler_params=pltpu.CompilerParams(
          dimension_semantics=("arbitrary",)),
  )(st_rm)


WSZ = 256
WGRID = B // WSZ


def _win_body(ids_row_ref, ids_col_ref, win_ref):
  eq = ids_col_ref[...] == ids_row_ref[...]
  pos = lax.broadcasted_iota(jnp.int32, (WSZ, B), 1)
  win_ref[0, 0, :] = jnp.max(jnp.where(eq, pos, -1), axis=1)


def _win_idx(ids_row, ids_col):
  return pl.pallas_call(
      _win_body,
      grid=(WGRID,),
      in_specs=[
          pl.BlockSpec((1, B), lambda i: (0, 0)),
          pl.BlockSpec((WSZ, 1), lambda i: (i, 0)),
      ],
      out_specs=pl.BlockSpec((1, 1, WSZ), lambda i: (i, 0, 0)),
      out_shape=jax.ShapeDtypeStruct((WGRID, 1, WSZ), jnp.int32),
      compiler_params=pltpu.CompilerParams(
          dimension_semantics=("arbitrary",)),
  )(ids_row, ids_col)


BSZ = 128
GRID = B // BSZ


def _ln(x, g, b):
  mu = jnp.mean(x, axis=-1, keepdims=True)
  var = jnp.mean((x - mu) * (x - mu), axis=-1, keepdims=True)
  return (x - mu) * lax.rsqrt(var + 1e-5) * g + b


def _l2norm(x):
  n = jnp.sqrt(jnp.sum(x * x, axis=-1, keepdims=True))
  return x / jnp.maximum(n, 1e-12)


def _dot_t(a, b):
  return lax.dot_general(a, b, (((1,), (1,)), ((), ())),
                         preferred_element_type=jnp.float32)


def _dot(a, b):
  return lax.dot_general(a, b, (((1,), (0,)), ((), ())),
                         preferred_element_type=jnp.float32)


def _tc_body(use_ref, item_ref, old_ref, rep_ref, exp_ref, seg_ref,
             Wq_ref, Wk_ref, Wv_ref, Wo_ref, bo_ref, wa_ref, ba_ref,
             we_ref, be_ref, W1_ref, b1_ref, W2_ref, b2_ref,
             g1_ref, c1_ref, g2_ref, c2_ref,
             ude_ref, new_ref):
  use = use_ref[...]
  item = item_ref[...]
  old = old_ref[...]
  rep = rep_ref[...]
  expm = exp_ref[...]
  seg = seg_ref[...]
  g1, c1 = g1_ref[...], c1_ref[...]
  g2, c2 = g2_ref[...], c2_ref[...]

  x_norm = _ln(use, g1, c1)
  q = _l2norm(_dot_t(x_norm, Wq_ref[...]))
  qrep = _dot(q, rep)
  read_content = _dot(old * qrep, seg)
  attn = _dot_t(read_content, Wo_ref[...]) + bo_ref[...]
  x = use + attn
  x2 = _ln(x, g2, c2)
  h = jnp.maximum(_dot_t(x2, W1_ref[...]) + b1_ref[...], 0.0)
  ffn = _dot_t(h, W2_ref[...]) + b2_ref[...]
  ude_ref[...] = x + ffn

  i_norm = _ln(item, g1, c1)
  k = _l2norm(_dot_t(i_norm, Wk_ref[...]))
  v = _dot_t(i_norm, Wv_ref[...])
  alpha = jax.nn.sigmoid(
      jnp.sum(i_norm * wa_ref[...], axis=1, keepdims=True) + ba_ref[0, 0])
  eta = jax.nn.sigmoid(
      jnp.sum(i_norm * we_ref[...], axis=1, keepdims=True) + be_ref[0, 0]
  ) * (D ** -0.5)
  krep = _dot(k, rep)
  pred = _dot(old * krep, seg)
  err = v - pred
  errrep = _dot(err, expm)
  new_ref[...] = (1.0 - alpha) * old + eta * (errrep * krep)


def _tc_dense(use, item, old2, rep, expm, seg, Wq, Wk, Wv, Wout, bout,
              w_alpha, b_alpha, w_eta, b_eta, W1, b1, W2, b2,
              ln1_g, ln1_b, ln2_g, ln2_b):
  full = lambda s: pl.BlockSpec(s, lambda i: tuple(0 for _ in s))
  return pl.pallas_call(
      _tc_body,
      grid=(GRID,),
      in_specs=[
          pl.BlockSpec((BSZ, D), lambda i: (i, 0)),
          pl.BlockSpec((BSZ, D), lambda i: (i, 0)),
          pl.BlockSpec((BSZ, DD), lambda i: (i, 0)),
          full((D, DD)), full((D, DD)), full((DD, D)),
          full((D, D)), full((D, D)), full((D, D)), full((D, D)),
          full((1, D)),
          full((1, D)), full((1, 1)),
          full((1, D)), full((1, 1)),
          full((DFF, D)), full((1, DFF)),
          full((D, DFF)), full((1, D)),
          full((1, D)), full((1, D)), full((1, D)), full((1, D)),
      ],
      out_specs=[
          pl.BlockSpec((BSZ, D), lambda i: (i, 0)),
          pl.BlockSpec((BSZ, DD), lambda i: (i, 0)),
      ],
      out_shape=[
          jax.ShapeDtypeStruct((B, D), jnp.float32),
          jax.ShapeDtypeStruct((B, DD), jnp.float32),
      ],
      compiler_params=pltpu.CompilerParams(
          dimension_semantics=("arbitrary",)),
  )(use, item, old2, rep, expm, seg, Wq, Wk, Wv, Wout, bout,
    w_alpha, b_alpha, w_eta, b_eta, W1, b1, W2, b2,
    ln1_g, ln1_b, ln2_g, ln2_b)


def kernel(user_ids, user_static_emb, item_emb, states, Wq, Wk, Wv, Wout, bout,
           w_alpha, b_alpha, w_eta, b_eta, W1, b1, W2, b2,
           ln1_g, ln1_b, ln2_g, ln2_b):
  ids = user_ids.astype(jnp.int32)
  st_t = states.transpose(1, 2, 0).reshape(DD, M)
  st_ref = jax.new_ref(_tc_transpose(st_t))

  old_flat = _sc_gather_ref(st_ref, ids)

  win3 = _win_idx(ids.reshape(1, B), ids.reshape(B, 1))

  pcol = lax.broadcasted_iota(jnp.int32, (D, DD), 1)
  prow = lax.broadcasted_iota(jnp.int32, (D, DD), 0)
  rep = (pcol % D == prow).astype(jnp.float32)
  expm = (pcol // D == prow).astype(jnp.float32)
  seg = expm.T

  ude, new2 = _tc_dense(
      user_static_emb, item_emb, old_flat, rep, expm, seg,
      Wq, Wk, Wv, Wout, bout.reshape(1, D),
      w_alpha, b_alpha.reshape(1, 1), w_eta, b_eta.reshape(1, 1),
      W1, b1.reshape(1, DFF), W2, b2.reshape(1, D),
      ln1_g.reshape(1, D), ln1_b.reshape(1, D),
      ln2_g.reshape(1, D), ln2_b.reshape(1, D))

  _sc_scatter(new2, ids, win3.reshape(B), st_ref)
  out_t = _tc_transpose_out(st_ref[...])
  return ude, out_t.reshape(D, D, M).transpose(2, 0, 1)
